# baseline (speedup 1.0000x reference)
import functools

import jax
import jax.numpy as jnp
from jax import lax
from jax.experimental import pallas as pl
from jax.experimental.pallas import tpu as pltpu
from jax.experimental.pallas import tpu_sc as plsc

N = 10000
D = 128
E = 320000
T = 2

NC = 2
NS = 16
NW = NC * NS

ECHUNK = 128
NBUF = 2
G = 16
CH = G * (-(-E // (NW * ECHUNK * G)))
NG = CH // G
E_PAD = NW * CH * ECHUNK
ROWS_PER_TILE = 8 * (-(-(N + 1) // (NS * 8)))
N_PAD = ROWS_PER_TILE * NS

NB = 1000
GRID = N // NB


def _h0_body(nt_ref, out_ref):
    nt = nt_ref[...]
    cols = lax.broadcasted_iota(jnp.int32, (NB, D), 1)
    out_ref[...] = (cols % 3 == nt).astype(jnp.float32)


def _flat_body(h_ref, wmT_ref, bm_ref, out_ref):
    h = h_ref[...]
    for t in range(4):
        out_ref[t] = (
            jnp.dot(h, wmT_ref[t], preferred_element_type=jnp.float32)
            + bm_ref[t]
        )


def _gru_body(h_ref, m0_ref, m1_ref, wihT_ref, whhT_ref, bih_ref, bhh_ref,
              out_ref):
    h = h_ref[...]
    m = m0_ref[...] + m1_ref[...]
    gi = jnp.dot(h, wihT_ref[...], preferred_element_type=jnp.float32) + bih_ref[...]
    gh = jnp.dot(m, whhT_ref[...], preferred_element_type=jnp.float32) + bhh_ref[...]
    r = jax.nn.sigmoid(gi[:, :D] + gh[:, :D])
    z = jax.nn.sigmoid(gi[:, D:2 * D] + gh[:, D:2 * D])
    ng = jnp.tanh(gi[:, 2 * D:] + r * gh[:, 2 * D:])
    out_ref[...] = (1.0 - z) * ng + z * m


def _heads_body(h0_ref, h_ref,
                fpW1T_ref, fpb1_ref, fpW2T_ref, fpb2_ref,
                gpW1T_ref, gpb1_ref, gpW2T_ref, gpb2_ref,
                fvW1T_ref, fvb1_ref, fvW2T_ref, fvb2_ref,
                gvW1T_ref, gvb1_ref, gvW2T_ref, gvb2_ref,
                p_ref, v_ref):
    i = pl.program_id(0)

    @pl.when(i == 0)
    def _():
        p_ref[...] = jnp.zeros_like(p_ref)
        v_ref[...] = jnp.zeros_like(v_ref)

    h0 = h0_ref[...]
    h = h_ref[...]
    h0h = jnp.concatenate([h0, h], axis=-1)

    def mlp(x, w1T, b1, w2T, b2):
        t1 = jnp.tanh(jnp.dot(x, w1T[...], preferred_element_type=jnp.float32)
                      + b1[...])
        return jnp.dot(t1, w2T[...], preferred_element_type=jnp.float32) + b2[...]

    fp = mlp(h0h, fpW1T_ref, fpb1_ref, fpW2T_ref, fpb2_ref)
    gp = mlp(h, gpW1T_ref, gpb1_ref, gpW2T_ref, gpb2_ref)
    fv = mlp(h0h, fvW1T_ref, fvb1_ref, fvW2T_ref, fvb2_ref)
    gv = mlp(h, gvW1T_ref, gvb1_ref, gvW2T_ref, gvb2_ref)

    p_ref[...] += jnp.sum(jax.nn.sigmoid(fp) * gp, axis=0, keepdims=True)
    v_ref[...] += jnp.sum(jax.nn.sigmoid(fv) * gv, axis=0, keepdims=True)

    @pl.when(i == GRID - 1)
    def _():
        p = p_ref[...]
        pmax = jnp.max(p, axis=-1, keepdims=True)
        lse = jnp.log(jnp.sum(jnp.exp(p - pmax), axis=-1, keepdims=True)) + pmax
        p_ref[...] = p - lse
        v_ref[...] = jnp.tanh(v_ref[...])


def _h0(node_types):
    return pl.pallas_call(
        _h0_body,
        grid=(GRID,),
        in_specs=[pl.BlockSpec((NB, 1), lambda i: (i, 0))],
        out_specs=pl.BlockSpec((NB, D), lambda i: (i, 0)),
        out_shape=jax.ShapeDtypeStruct((N, D), jnp.float32),
    )(node_types.reshape(N, 1))


def _flat(h, wmT, bm2):
    return pl.pallas_call(
        _flat_body,
        grid=(GRID,),
        in_specs=[
            pl.BlockSpec((NB, D), lambda i: (i, 0)),
            pl.BlockSpec((4, D, D), lambda i: (0, 0, 0)),
            pl.BlockSpec((4, 1, D), lambda i: (0, 0, 0)),
        ],
        out_specs=pl.BlockSpec((4, NB, D), lambda i: (0, i, 0)),
        out_shape=jax.ShapeDtypeStruct((4, N, D), jnp.float32),
    )(h, wmT, bm2)


def _gru(h, parts, wihT, whhT, bih2, bhh2):
    return pl.pallas_call(
        _gru_body,
        grid=(GRID,),
        in_specs=[
            pl.BlockSpec((NB, D), lambda i: (i, 0)),
            pl.BlockSpec((NB, D), lambda i: (i, 0)),
            pl.BlockSpec((NB, D), lambda i: (i, 0)),
            pl.BlockSpec((D, 3 * D), lambda i: (0, 0)),
            pl.BlockSpec((D, 3 * D), lambda i: (0, 0)),
            pl.BlockSpec((1, 3 * D), lambda i: (0, 0)),
            pl.BlockSpec((1, 3 * D), lambda i: (0, 0)),
        ],
        out_specs=pl.BlockSpec((NB, D), lambda i: (i, 0)),
        out_shape=jax.ShapeDtypeStruct((N, D), jnp.float32),
    )(h, parts[0, :N], parts[1, :N], wihT, whhT, bih2, bhh2)


def _heads(h0, h, fpW1T, fpb1, fpW2T, fpb2, gpW1T, gpb1, gpW2T, gpb2,
           fvW1T, fvb1, fvW2T, fvb2, gvW1T, gvb1, gvW2T, gvb2):
    def full(shape):
        return pl.BlockSpec(shape, lambda i: tuple(0 for _ in shape))
    return pl.pallas_call(
        _heads_body,
        grid=(GRID,),
        in_specs=[
            pl.BlockSpec((NB, D), lambda i: (i, 0)),
            pl.BlockSpec((NB, D), lambda i: (i, 0)),
            full((2 * D, D)), full((1, D)), full((D, 2)), full((1, 2)),
            full((D, D)), full((1, D)), full((D, 2)), full((1, 2)),
            full((2 * D, D)), full((1, D)), full((D, 1)), full((1, 1)),
            full((D, D)), full((1, D)), full((D, 1)), full((1, 1)),
        ],
        out_specs=[
            pl.BlockSpec((1, 2), lambda i: (0, 0)),
            pl.BlockSpec((1, 1), lambda i: (0, 0)),
        ],
        out_shape=[
            jax.ShapeDtypeStruct((1, 2), jnp.float32),
            jax.ShapeDtypeStruct((1, 1), jnp.float32),
        ],
    )(h0, h, fpW1T, fpb1, fpW2T, fpb2, gpW1T, gpb1, gpW2T, gpb2,
      fvW1T, fvb1, fvW2T, fvb2, gvW1T, gvb1, gvW2T, gvb2)


def _sc_edge_body(flat_hbm, gidx_hbm, dst_hbm, zeros_hbm, out_hbm,
                  gidx_v, dst_v, rows_a, rows_b,
                  sg0, sg1, sem_i, acc):
    rows = (rows_a, rows_b)
    sg = (sg0, sg1)
    c = lax.axis_index("c")
    s = lax.axis_index("s")
    wid = s * NC + c
    base = s * ROWS_PER_TILE

    pltpu.sync_copy(zeros_hbm, acc.at[pl.ds(base, ROWS_PER_TILE)])
    pltpu.sync_copy(gidx_hbm.at[wid, pl.ds(0, G)], gidx_v.at[0])
    pltpu.sync_copy(dst_hbm.at[wid, pl.ds(0, G)], dst_v.at[0])
    pltpu.async_copy(gidx_hbm.at[wid, pl.ds(G, G)], gidx_v.at[1], sem_i)
    pltpu.async_copy(dst_hbm.at[wid, pl.ds(G, G)], dst_v.at[1], sem_i)
    plsc.subcore_barrier()

    def body(i, carry):
        j = i * NBUF
        for b in range(NBUF):
            jj = j + b
            bn = (b + 1) % NBUF
            jn = jj + 1

            @pl.when(jnp.logical_and((jn & (G - 1)) == 0, jn < CH))
            def _():
                gn = jn // G
                pltpu.make_async_copy(gidx_hbm.at[wid, pl.ds(0, G)],
                                      gidx_v.at[0], sem_i).wait()
                pltpu.make_async_copy(dst_hbm.at[wid, pl.ds(0, G)],
                                      dst_v.at[0], sem_i).wait()

                @pl.when(gn + 1 < NG)
                def _():
                    pltpu.async_copy(gidx_hbm.at[wid, pl.ds((gn + 1) * G, G)],
                                     gidx_v.at[(gn + 1) % 2], sem_i)
                    pltpu.async_copy(dst_hbm.at[wid, pl.ds((gn + 1) * G, G)],
                                     dst_v.at[(gn + 1) % 2], sem_i)

            pltpu.async_copy(
                flat_hbm.at[gidx_v.at[(jj // G) % 2, jj & (G - 1)]],
                rows[b], sg[b]).wait()
            pltpu.sync_copy(rows[b],
                            acc.at[dst_v.at[(jj // G) % 2, jj & (G - 1)]],
                            add=True)
        return carry

    lax.fori_loop(0, CH // NBUF, body, 0)
    plsc.subcore_barrier()

    pltpu.sync_copy(acc.at[pl.ds(base, ROWS_PER_TILE)],
                    out_hbm.at[c, pl.ds(base, ROWS_PER_TILE)])


@functools.cache
def _sc_edge_kernel():
    return pl.kernel(
        _sc_edge_body,
        out_type=jax.ShapeDtypeStruct((NC, N_PAD, D), jnp.float32),
        mesh=plsc.VectorSubcoreMesh(core_axis_name="c", subcore_axis_name="s",
                                    num_cores=NC, num_subcores=NS),
        scratch_types=[
            pltpu.VMEM((2, G, ECHUNK), jnp.int32),
            pltpu.VMEM((2, G, ECHUNK), jnp.int32),
            pltpu.VMEM((ECHUNK, D), jnp.float32),
            pltpu.VMEM((ECHUNK, D), jnp.float32),
        ] + [pltpu.SemaphoreType.DMA] * 3 + [
            pltpu.VMEM_SHARED((N_PAD, D), jnp.float32),
        ],
    )


def _sc_edge(flat, gidx3, dst3, zrows):
    return _sc_edge_kernel()(flat, gidx3, dst3, zrows)


def kernel(node_types, edge_index, edge_type, Wm, bm, W_ih, W_hh, b_ih, b_hh,
           fp_W1, fp_b1, fp_W2, fp_b2, gp_W1, gp_b1, gp_W2, gp_b2,
           fv_W1, fv_b1, fv_W2, fv_b2, gv_W1, gv_b1, gv_W2, gv_b2):
    node_types = node_types.astype(jnp.int32)
    src = edge_index[0].astype(jnp.int32)
    dst = edge_index[1].astype(jnp.int32)
    et = edge_type.astype(jnp.int32)

    gidx = et * N + src
    gidx = jnp.concatenate([gidx, jnp.zeros((E_PAD - E,), jnp.int32)])
    dstp = jnp.concatenate([dst, jnp.full((E_PAD - E,), N, jnp.int32)])
    gidx3 = gidx.reshape(NW, CH, ECHUNK)
    dst3 = dstp.reshape(NW, CH, ECHUNK)
    zrows = jnp.zeros((ROWS_PER_TILE, D), jnp.float32)

    wmT = jnp.transpose(Wm, (0, 2, 1))
    bm2 = bm.reshape(4, 1, D)
    wihT = W_ih.T
    whhT = W_hh.T
    bih2 = b_ih.reshape(1, 3 * D)
    bhh2 = b_hh.reshape(1, 3 * D)

    h0 = _h0(node_types)
    h = h0
    for _ in range(T):
        flat = _flat(h, wmT, bm2).reshape(4 * N, D)
        parts = _sc_edge(flat, gidx3, dst3, zrows)
        h = _gru(h, parts, wihT, whhT, bih2, bhh2)

    p, v = _heads(h0, h,
                  fp_W1.T, fp_b1.reshape(1, D), fp_W2.T, fp_b2.reshape(1, 2),
                  gp_W1.T, gp_b1.reshape(1, D), gp_W2.T, gp_b2.reshape(1, 2),
                  fv_W1.T, fv_b1.reshape(1, D), fv_W2.T, fv_b2.reshape(1, 1),
                  gv_W1.T, gv_b1.reshape(1, D), gv_W2.T, gv_b2.reshape(1, 1))
    return p.reshape(2), v.reshape(1)


# speedup vs baseline: 1.1070x; 1.1070x over previous
import functools

import jax
import jax.numpy as jnp
from jax import lax
from jax.experimental import pallas as pl
from jax.experimental.pallas import tpu as pltpu
from jax.experimental.pallas import tpu_sc as plsc

N = 10000
D = 128
E = 320000
T = 2

NC = 2
NS = 16
NW = NC * NS

ECHUNK = 128
NBUF = 2
G = 16
CH = G * (-(-E // (NW * ECHUNK * G)))
NG = CH // G
E_PAD = NW * CH * ECHUNK
ROWS_PER_TILE = 8 * (-(-(N + 1) // (NS * 8)))
N_PAD = ROWS_PER_TILE * NS

NB = 1000
GRID = N // NB


def _h0_body(nt_ref, out_ref):
    nt = nt_ref[...]
    cols = lax.broadcasted_iota(jnp.int32, (NB, D), 1)
    out_ref[...] = (cols % 3 == nt).astype(jnp.float32)


def _flat_body(h_ref, wmT_ref, bm_ref, out_ref):
    h = h_ref[...]
    for t in range(4):
        out_ref[t] = (
            jnp.dot(h, wmT_ref[t], preferred_element_type=jnp.float32)
            + bm_ref[t]
        )


def _gru_body(h_ref, m0_ref, m1_ref, wihT_ref, whhT_ref, bih_ref, bhh_ref,
              out_ref):
    h = h_ref[...]
    m = m0_ref[...] + m1_ref[...]
    gi = jnp.dot(h, wihT_ref[...], preferred_element_type=jnp.float32) + bih_ref[...]
    gh = jnp.dot(m, whhT_ref[...], preferred_element_type=jnp.float32) + bhh_ref[...]
    r = jax.nn.sigmoid(gi[:, :D] + gh[:, :D])
    z = jax.nn.sigmoid(gi[:, D:2 * D] + gh[:, D:2 * D])
    ng = jnp.tanh(gi[:, 2 * D:] + r * gh[:, 2 * D:])
    out_ref[...] = (1.0 - z) * ng + z * m


def _heads_body(h0_ref, h_ref,
                fpW1T_ref, fpb1_ref, fpW2T_ref, fpb2_ref,
                gpW1T_ref, gpb1_ref, gpW2T_ref, gpb2_ref,
                fvW1T_ref, fvb1_ref, fvW2T_ref, fvb2_ref,
                gvW1T_ref, gvb1_ref, gvW2T_ref, gvb2_ref,
                p_ref, v_ref):
    i = pl.program_id(0)

    @pl.when(i == 0)
    def _():
        p_ref[...] = jnp.zeros_like(p_ref)
        v_ref[...] = jnp.zeros_like(v_ref)

    h0 = h0_ref[...]
    h = h_ref[...]
    h0h = jnp.concatenate([h0, h], axis=-1)

    def mlp(x, w1T, b1, w2T, b2):
        t1 = jnp.tanh(jnp.dot(x, w1T[...], preferred_element_type=jnp.float32)
                      + b1[...])
        return jnp.dot(t1, w2T[...], preferred_element_type=jnp.float32) + b2[...]

    fp = mlp(h0h, fpW1T_ref, fpb1_ref, fpW2T_ref, fpb2_ref)
    gp = mlp(h, gpW1T_ref, gpb1_ref, gpW2T_ref, gpb2_ref)
    fv = mlp(h0h, fvW1T_ref, fvb1_ref, fvW2T_ref, fvb2_ref)
    gv = mlp(h, gvW1T_ref, gvb1_ref, gvW2T_ref, gvb2_ref)

    p_ref[...] += jnp.sum(jax.nn.sigmoid(fp) * gp, axis=0, keepdims=True)
    v_ref[...] += jnp.sum(jax.nn.sigmoid(fv) * gv, axis=0, keepdims=True)

    @pl.when(i == GRID - 1)
    def _():
        p = p_ref[...]
        pmax = jnp.max(p, axis=-1, keepdims=True)
        lse = jnp.log(jnp.sum(jnp.exp(p - pmax), axis=-1, keepdims=True)) + pmax
        p_ref[...] = p - lse
        v_ref[...] = jnp.tanh(v_ref[...])


def _h0(node_types):
    return pl.pallas_call(
        _h0_body,
        grid=(GRID,),
        in_specs=[pl.BlockSpec((NB, 1), lambda i: (i, 0))],
        out_specs=pl.BlockSpec((NB, D), lambda i: (i, 0)),
        out_shape=jax.ShapeDtypeStruct((N, D), jnp.float32),
    )(node_types.reshape(N, 1))


def _flat(h, wmT, bm2):
    return pl.pallas_call(
        _flat_body,
        grid=(GRID,),
        in_specs=[
            pl.BlockSpec((NB, D), lambda i: (i, 0)),
            pl.BlockSpec((4, D, D), lambda i: (0, 0, 0)),
            pl.BlockSpec((4, 1, D), lambda i: (0, 0, 0)),
        ],
        out_specs=pl.BlockSpec((4, NB, D), lambda i: (0, i, 0)),
        out_shape=jax.ShapeDtypeStruct((4, N, D), jnp.float32),
    )(h, wmT, bm2)


def _gru(h, parts, wihT, whhT, bih2, bhh2):
    return pl.pallas_call(
        _gru_body,
        grid=(GRID,),
        in_specs=[
            pl.BlockSpec((NB, D), lambda i: (i, 0)),
            pl.BlockSpec((NB, D), lambda i: (i, 0)),
            pl.BlockSpec((NB, D), lambda i: (i, 0)),
            pl.BlockSpec((D, 3 * D), lambda i: (0, 0)),
            pl.BlockSpec((D, 3 * D), lambda i: (0, 0)),
            pl.BlockSpec((1, 3 * D), lambda i: (0, 0)),
            pl.BlockSpec((1, 3 * D), lambda i: (0, 0)),
        ],
        out_specs=pl.BlockSpec((NB, D), lambda i: (i, 0)),
        out_shape=jax.ShapeDtypeStruct((N, D), jnp.float32),
    )(h, parts[0, :N], parts[1, :N], wihT, whhT, bih2, bhh2)


def _heads(h0, h, fpW1T, fpb1, fpW2T, fpb2, gpW1T, gpb1, gpW2T, gpb2,
           fvW1T, fvb1, fvW2T, fvb2, gvW1T, gvb1, gvW2T, gvb2):
    def full(shape):
        return pl.BlockSpec(shape, lambda i: tuple(0 for _ in shape))
    return pl.pallas_call(
        _heads_body,
        grid=(GRID,),
        in_specs=[
            pl.BlockSpec((NB, D), lambda i: (i, 0)),
            pl.BlockSpec((NB, D), lambda i: (i, 0)),
            full((2 * D, D)), full((1, D)), full((D, 2)), full((1, 2)),
            full((D, D)), full((1, D)), full((D, 2)), full((1, 2)),
            full((2 * D, D)), full((1, D)), full((D, 1)), full((1, 1)),
            full((D, D)), full((1, D)), full((D, 1)), full((1, 1)),
        ],
        out_specs=[
            pl.BlockSpec((1, 2), lambda i: (0, 0)),
            pl.BlockSpec((1, 1), lambda i: (0, 0)),
        ],
        out_shape=[
            jax.ShapeDtypeStruct((1, 2), jnp.float32),
            jax.ShapeDtypeStruct((1, 1), jnp.float32),
        ],
    )(h0, h, fpW1T, fpb1, fpW2T, fpb2, gpW1T, gpb1, gpW2T, gpb2,
      fvW1T, fvb1, fvW2T, fvb2, gvW1T, gvb1, gvW2T, gvb2)


def _sc_edge_body(flat_hbm, packed_hbm, zeros_hbm, out_hbm,
                  packed_v, gu0, gu1, du0, du1, rows_a, rows_b,
                  sg0, sg1, acc):
    rows = (rows_a, rows_b)
    gu = (gu0, gu1)
    du = (du0, du1)
    sg = (sg0, sg1)
    c = lax.axis_index("c")
    s = lax.axis_index("s")
    wid = s * NC + c
    base = s * ROWS_PER_TILE

    pltpu.sync_copy(zeros_hbm, acc.at[pl.ds(base, ROWS_PER_TILE)])
    pltpu.sync_copy(packed_hbm.at[wid], packed_v)
    plsc.subcore_barrier()

    def unpack(jc, b):
        for k in range(8):
            pv = packed_v[jc, pl.ds(16 * k, 16)]
            gu[b][0, pl.ds(16 * k, 16)] = pv & 0xFFFF
            du[b][0, pl.ds(16 * k, 16)] = lax.shift_right_logical(pv, 16)

    unpack(0, 0)
    pltpu.async_copy(flat_hbm.at[gu[0].at[0]], rows[0], sg[0])

    def step(jj, b):
        bn = (b + 1) % NBUF
        unpack(jj + 1, bn)
        pltpu.async_copy(flat_hbm.at[gu[bn].at[0]], rows[bn], sg[bn])
        pltpu.make_async_copy(flat_hbm.at[gu[b].at[0]], rows[b],
                              sg[b]).wait()
        pltpu.sync_copy(rows[b], acc.at[du[b].at[0]], add=True)

    def body(i, carry):
        step(i * NBUF, 0)
        step(i * NBUF + 1, 1)
        return carry

    lax.fori_loop(0, CH // NBUF - 1, body, 0)
    step(CH - 2, 0)
    pltpu.make_async_copy(flat_hbm.at[gu[1].at[0]], rows[1], sg[1]).wait()
    pltpu.sync_copy(rows[1], acc.at[du[1].at[0]], add=True)
    plsc.subcore_barrier()

    pltpu.sync_copy(acc.at[pl.ds(base, ROWS_PER_TILE)],
                    out_hbm.at[c, pl.ds(base, ROWS_PER_TILE)])


@functools.cache
def _sc_edge_kernel():
    return pl.kernel(
        _sc_edge_body,
        out_type=jax.ShapeDtypeStruct((NC, N_PAD, D), jnp.float32),
        mesh=plsc.VectorSubcoreMesh(core_axis_name="c", subcore_axis_name="s",
                                    num_cores=NC, num_subcores=NS),
        scratch_types=[
            pltpu.VMEM((CH, ECHUNK), jnp.int32),
            pltpu.VMEM((1, ECHUNK), jnp.int32),
            pltpu.VMEM((1, ECHUNK), jnp.int32),
            pltpu.VMEM((1, ECHUNK), jnp.int32),
            pltpu.VMEM((1, ECHUNK), jnp.int32),
            pltpu.VMEM((ECHUNK, D), jnp.float32),
            pltpu.VMEM((ECHUNK, D), jnp.float32),
        ] + [pltpu.SemaphoreType.DMA] * 2 + [
            pltpu.VMEM_SHARED((N_PAD, D), jnp.float32),
        ],
    )


def _sc_edge(flat, packed3, zrows):
    return _sc_edge_kernel()(flat, packed3, zrows)


def kernel(node_types, edge_index, edge_type, Wm, bm, W_ih, W_hh, b_ih, b_hh,
           fp_W1, fp_b1, fp_W2, fp_b2, gp_W1, gp_b1, gp_W2, gp_b2,
           fv_W1, fv_b1, fv_W2, fv_b2, gv_W1, gv_b1, gv_W2, gv_b2):
    node_types = node_types.astype(jnp.int32)
    src = edge_index[0].astype(jnp.int32)
    dst = edge_index[1].astype(jnp.int32)
    et = edge_type.astype(jnp.int32)

    packed = (et * N + src) | (dst << 16)
    packed = jnp.concatenate(
        [packed, jnp.full((E_PAD - E,), N << 16, jnp.int32)])
    packed3 = packed.reshape(NW, CH, ECHUNK)
    zrows = jnp.zeros((ROWS_PER_TILE, D), jnp.float32)

    wmT = jnp.transpose(Wm, (0, 2, 1))
    bm2 = bm.reshape(4, 1, D)
    wihT = W_ih.T
    whhT = W_hh.T
    bih2 = b_ih.reshape(1, 3 * D)
    bhh2 = b_hh.reshape(1, 3 * D)

    h0 = _h0(node_types)
    h = h0
    for _ in range(T):
        flat = _flat(h, wmT, bm2).reshape(4 * N, D)
        parts = _sc_edge(flat, packed3, zrows)
        h = _gru(h, parts, wihT, whhT, bih2, bhh2)

    p, v = _heads(h0, h,
                  fp_W1.T, fp_b1.reshape(1, D), fp_W2.T, fp_b2.reshape(1, 2),
                  gp_W1.T, gp_b1.reshape(1, D), gp_W2.T, gp_b2.reshape(1, 2),
                  fv_W1.T, fv_b1.reshape(1, D), fv_W2.T, fv_b2.reshape(1, 1),
                  gv_W1.T, gv_b1.reshape(1, D), gv_W2.T, gv_b2.reshape(1, 1))
    return p.reshape(2), v.reshape(1)


# speedup vs baseline: 1.7887x; 1.6159x over previous
import functools

import jax
import jax.numpy as jnp
from jax import lax
from jax.experimental import pallas as pl
from jax.experimental.pallas import tpu as pltpu
from jax.experimental.pallas import tpu_sc as plsc

N = 10000
D = 128
E = 320000
T = 2

NC = 2
NS = 16
NW = NC * NS

ECHUNK = 128
CHF = 102
CHS = 56
CHMAX = CHF
FAST_CORE = 1
E_PAD = NS * (CHF + CHS) * ECHUNK
ROWS_PER_TILE = 8 * (-(-(N + 1) // (NS * 8)))
N_PAD = ROWS_PER_TILE * NS

NB = 1000
GRID = N // NB


def _h0_body(nt_ref, out_ref):
    nt = nt_ref[...]
    cols = lax.broadcasted_iota(jnp.int32, (NB, D), 1)
    out_ref[...] = (cols % 3 == nt).astype(jnp.float32)


def _flat_body(h_ref, wmT_ref, bm_ref, out_ref):
    h = h_ref[...]
    for t in range(4):
        out_ref[t] = (
            jnp.dot(h, wmT_ref[t], preferred_element_type=jnp.float32)
            + bm_ref[t]
        )


def _gru_body(h_ref, m0_ref, m1_ref, wihT_ref, whhT_ref, bih_ref, bhh_ref,
              out_ref):
    h = h_ref[...]
    m = m0_ref[...] + m1_ref[...]
    gi = jnp.dot(h, wihT_ref[...], preferred_element_type=jnp.float32) + bih_ref[...]
    gh = jnp.dot(m, whhT_ref[...], preferred_element_type=jnp.float32) + bhh_ref[...]
    r = jax.nn.sigmoid(gi[:, :D] + gh[:, :D])
    z = jax.nn.sigmoid(gi[:, D:2 * D] + gh[:, D:2 * D])
    ng = jnp.tanh(gi[:, 2 * D:] + r * gh[:, 2 * D:])
    out_ref[...] = (1.0 - z) * ng + z * m


def _heads_body(h0_ref, h_ref,
                fpW1T_ref, fpb1_ref, fpW2T_ref, fpb2_ref,
                gpW1T_ref, gpb1_ref, gpW2T_ref, gpb2_ref,
                fvW1T_ref, fvb1_ref, fvW2T_ref, fvb2_ref,
                gvW1T_ref, gvb1_ref, gvW2T_ref, gvb2_ref,
                p_ref, v_ref):
    i = pl.program_id(0)

    @pl.when(i == 0)
    def _():
        p_ref[...] = jnp.zeros_like(p_ref)
        v_ref[...] = jnp.zeros_like(v_ref)

    h0 = h0_ref[...]
    h = h_ref[...]
    h0h = jnp.concatenate([h0, h], axis=-1)

    def mlp(x, w1T, b1, w2T, b2):
        t1 = jnp.tanh(jnp.dot(x, w1T[...], preferred_element_type=jnp.float32)
                      + b1[...])
        return jnp.dot(t1, w2T[...], preferred_element_type=jnp.float32) + b2[...]

    fp = mlp(h0h, fpW1T_ref, fpb1_ref, fpW2T_ref, fpb2_ref)
    gp = mlp(h, gpW1T_ref, gpb1_ref, gpW2T_ref, gpb2_ref)
    fv = mlp(h0h, fvW1T_ref, fvb1_ref, fvW2T_ref, fvb2_ref)
    gv = mlp(h, gvW1T_ref, gvb1_ref, gvW2T_ref, gvb2_ref)

    p_ref[...] += jnp.sum(jax.nn.sigmoid(fp) * gp, axis=0, keepdims=True)
    v_ref[...] += jnp.sum(jax.nn.sigmoid(fv) * gv, axis=0, keepdims=True)

    @pl.when(i == GRID - 1)
    def _():
        p = p_ref[...]
        pmax = jnp.max(p, axis=-1, keepdims=True)
        lse = jnp.log(jnp.sum(jnp.exp(p - pmax), axis=-1, keepdims=True)) + pmax
        p_ref[...] = p - lse
        v_ref[...] = jnp.tanh(v_ref[...])


def _h0(node_types):
    return pl.pallas_call(
        _h0_body,
        grid=(GRID,),
        in_specs=[pl.BlockSpec((NB, 1), lambda i: (i, 0))],
        out_specs=pl.BlockSpec((NB, D), lambda i: (i, 0)),
        out_shape=jax.ShapeDtypeStruct((N, D), jnp.float32),
    )(node_types.reshape(N, 1))


def _flat(h, wmT, bm2):
    return pl.pallas_call(
        _flat_body,
        grid=(GRID,),
        in_specs=[
            pl.BlockSpec((NB, D), lambda i: (i, 0)),
            pl.BlockSpec((4, D, D), lambda i: (0, 0, 0)),
            pl.BlockSpec((4, 1, D), lambda i: (0, 0, 0)),
        ],
        out_specs=pl.BlockSpec((4, NB, D), lambda i: (0, i, 0)),
        out_shape=jax.ShapeDtypeStruct((4, N, D), jnp.float32),
    )(h, wmT, bm2)


def _gru(h, parts, wihT, whhT, bih2, bhh2):
    return pl.pallas_call(
        _gru_body,
        grid=(GRID,),
        in_specs=[
            pl.BlockSpec((NB, D), lambda i: (i, 0)),
            pl.BlockSpec((NB, D), lambda i: (i, 0)),
            pl.BlockSpec((NB, D), lambda i: (i, 0)),
            pl.BlockSpec((D, 3 * D), lambda i: (0, 0)),
            pl.BlockSpec((D, 3 * D), lambda i: (0, 0)),
            pl.BlockSpec((1, 3 * D), lambda i: (0, 0)),
            pl.BlockSpec((1, 3 * D), lambda i: (0, 0)),
        ],
        out_specs=pl.BlockSpec((NB, D), lambda i: (i, 0)),
        out_shape=jax.ShapeDtypeStruct((N, D), jnp.float32),
    )(h, parts[0, :N], parts[1, :N], wihT, whhT, bih2, bhh2)


def _heads(h0, h, fpW1T, fpb1, fpW2T, fpb2, gpW1T, gpb1, gpW2T, gpb2,
           fvW1T, fvb1, fvW2T, fvb2, gvW1T, gvb1, gvW2T, gvb2):
    def full(shape):
        return pl.BlockSpec(shape, lambda i: tuple(0 for _ in shape))
    return pl.pallas_call(
        _heads_body,
        grid=(GRID,),
        in_specs=[
            pl.BlockSpec((NB, D), lambda i: (i, 0)),
            pl.BlockSpec((NB, D), lambda i: (i, 0)),
            full((2 * D, D)), full((1, D)), full((D, 2)), full((1, 2)),
            full((D, D)), full((1, D)), full((D, 2)), full((1, 2)),
            full((2 * D, D)), full((1, D)), full((D, 1)), full((1, 1)),
            full((D, D)), full((1, D)), full((D, 1)), full((1, 1)),
        ],
        out_specs=[
            pl.BlockSpec((1, 2), lambda i: (0, 0)),
            pl.BlockSpec((1, 1), lambda i: (0, 0)),
        ],
        out_shape=[
            jax.ShapeDtypeStruct((1, 2), jnp.float32),
            jax.ShapeDtypeStruct((1, 1), jnp.float32),
        ],
    )(h0, h, fpW1T, fpb1, fpW2T, fpb2, gpW1T, gpb1, gpW2T, gpb2,
      fvW1T, fvb1, fvW2T, fvb2, gvW1T, gvb1, gvW2T, gvb2)


def _sc_edge_body(flat_hbm, gidx_hbm, dst_hbm, zeros_hbm, out_hbm,
                  gidx_v, dst_v, rows_v, sem, acc):
    c = lax.axis_index("c")
    s = lax.axis_index("s")
    wid = s * NC + c
    base = s * ROWS_PER_TILE

    pltpu.sync_copy(zeros_hbm, acc.at[pl.ds(base, ROWS_PER_TILE)])
    pltpu.sync_copy(gidx_hbm.at[wid], gidx_v)
    pltpu.sync_copy(dst_hbm.at[wid], dst_v)
    plsc.subcore_barrier()

    nch = jnp.where(c == FAST_CORE, CHF, CHS)

    def body(j, carry):
        pltpu.async_copy(flat_hbm.at[gidx_v.at[j]], rows_v, sem).wait()
        pltpu.sync_copy(rows_v, acc.at[dst_v.at[j]], add=True)
        return carry

    lax.fori_loop(0, nch, body, 0)
    plsc.subcore_barrier()

    pltpu.sync_copy(acc.at[pl.ds(base, ROWS_PER_TILE)],
                    out_hbm.at[c, pl.ds(base, ROWS_PER_TILE)])


@functools.cache
def _sc_edge_kernel():
    return pl.kernel(
        _sc_edge_body,
        out_type=jax.ShapeDtypeStruct((NC, N_PAD, D), jnp.float32),
        mesh=plsc.VectorSubcoreMesh(core_axis_name="c", subcore_axis_name="s",
                                    num_cores=NC, num_subcores=NS),
        scratch_types=[
            pltpu.VMEM((CHMAX, ECHUNK), jnp.int32),
            pltpu.VMEM((CHMAX, ECHUNK), jnp.int32),
            pltpu.VMEM((ECHUNK, D), jnp.float32),
            pltpu.SemaphoreType.DMA,
            pltpu.VMEM_SHARED((N_PAD, D), jnp.float32),
        ],
    )


def _sc_edge(flat, gidx3, dst3, zrows):
    return _sc_edge_kernel()(flat, gidx3, dst3, zrows)


def kernel(node_types, edge_index, edge_type, Wm, bm, W_ih, W_hh, b_ih, b_hh,
           fp_W1, fp_b1, fp_W2, fp_b2, gp_W1, gp_b1, gp_W2, gp_b2,
           fv_W1, fv_b1, fv_W2, fv_b2, gv_W1, gv_b1, gv_W2, gv_b2):
    node_types = node_types.astype(jnp.int32)
    src = edge_index[0].astype(jnp.int32)
    dst = edge_index[1].astype(jnp.int32)
    et = edge_type.astype(jnp.int32)

    def split_uneven(x, fill):
        x = jnp.concatenate([x, jnp.full((E_PAD - E,), fill, jnp.int32)])
        blk = x.reshape(NS, (CHF + CHS) * ECHUNK)
        big = blk[:, :CHF * ECHUNK].reshape(NS, CHF, ECHUNK)
        sml = blk[:, CHF * ECHUNK:].reshape(NS, CHS, ECHUNK)
        sml = jnp.pad(sml, ((0, 0), (0, CHMAX - CHS), (0, 0)),
                      constant_values=fill)
        pair = [big, sml] if FAST_CORE == 0 else [sml, big]
        return jnp.stack(pair, axis=1).reshape(NW, CHMAX, ECHUNK)

    gidx3 = split_uneven(et * N + src, 0)
    dst3 = split_uneven(dst, N)
    zrows = jnp.zeros((ROWS_PER_TILE, D), jnp.float32)

    wmT = jnp.transpose(Wm, (0, 2, 1))
    bm2 = bm.reshape(4, 1, D)
    wihT = W_ih.T
    whhT = W_hh.T
    bih2 = b_ih.reshape(1, 3 * D)
    bhh2 = b_hh.reshape(1, 3 * D)

    h0 = _h0(node_types)
    h = h0
    for _ in range(T):
        flat = _flat(h, wmT, bm2).reshape(4 * N, D)
        parts = _sc_edge(flat, gidx3, dst3, zrows)
        h = _gru(h, parts, wihT, whhT, bih2, bhh2)

    p, v = _heads(h0, h,
                  fp_W1.T, fp_b1.reshape(1, D), fp_W2.T, fp_b2.reshape(1, 2),
                  gp_W1.T, gp_b1.reshape(1, D), gp_W2.T, gp_b2.reshape(1, 2),
                  fv_W1.T, fv_b1.reshape(1, D), fv_W2.T, fv_b2.reshape(1, 1),
                  gv_W1.T, gv_b1.reshape(1, D), gv_W2.T, gv_b2.reshape(1, 1))
    return p.reshape(2), v.reshape(1)
